# baseline (speedup 1.0000x reference)
import functools

import numpy as np

import jax
import jax.numpy as jnp
from jax import lax
from jax.experimental import pallas as pl
from jax.experimental.pallas import tpu as pltpu
from jax.experimental.pallas import tpu_sc as plsc

_B, _H, _W = 8, 512, 512
_HW = _H * _W
_N_MATCHED = 4096
_N_UNMATCHED = 1024

_NW = 32
_CHUNK = 128
_LANES = 16
_LIN = 2 * _B * _HW

_N_M = 2 * _B * _N_MATCHED
_N_U = _B * _N_UNMATCHED
_M_PER_W = _N_M // _NW
_U_PER_W = _N_U // _NW
_PER_W = _M_PER_W + _U_PER_W
_CHUNKS = _PER_W // _CHUNK

_ROWS_PER_W = 2 * _B * _H // _NW


def _sigmoid16(x):
    return 1.0 / (1.0 + jnp.exp(-x))


@functools.partial(
    pl.kernel,
    out_type=(jax.ShapeDtypeStruct((_NW, _LANES), jnp.float32),
              jax.ShapeDtypeStruct((_LIN,), jnp.float32)),
    mesh=plsc.VectorSubcoreMesh(core_axis_name="c", subcore_axis_name="s"),
    scratch_types=[
        pltpu.VMEM((_PER_W,), jnp.int32),
        pltpu.VMEM((_PER_W,), jnp.int32),
        pltpu.VMEM((2 * _PER_W,), jnp.float32),
        pltpu.SemaphoreType.DMA,
        pltpu.SemaphoreType.DMA,
        pltpu.SemaphoreType.DMA,
    ],
)
def _bm_loss_sc(inp_hbm, tgt_hbm, am_hbm, bm_hbm, au_hbm, bu_hbm,
                out_hbm, lin_hbm, ia_v, ib_v, vab_v, isem, gsem, lsem):
    c = lax.axis_index("c")
    s = lax.axis_index("s")
    wid = s * 2 + c

    i1 = pltpu.async_copy(am_hbm.at[pl.ds(wid * _M_PER_W, _M_PER_W)],
                          ia_v.at[pl.ds(0, _M_PER_W)], isem)
    i2 = pltpu.async_copy(bm_hbm.at[pl.ds(wid * _M_PER_W, _M_PER_W)],
                          ib_v.at[pl.ds(0, _M_PER_W)], isem)
    i3 = pltpu.async_copy(au_hbm.at[pl.ds(wid * _U_PER_W, _U_PER_W)],
                          ia_v.at[pl.ds(_M_PER_W, _U_PER_W)], isem)
    i4 = pltpu.async_copy(bu_hbm.at[pl.ds(wid * _U_PER_W, _U_PER_W)],
                          ib_v.at[pl.ds(_M_PER_W, _U_PER_W)], isem)

    fld = s >> 3
    b = 4 * c + ((s >> 1) & 3)
    r0 = (s & 1) * (_H // 2)
    base = fld * (_B * _HW) + b * _HW + r0 * _W

    def lin_rows(src_ref):
        def body(r, carry):
            pltpu.async_copy(src_ref.at[b, 0, r0 + r, :],
                             lin_hbm.at[pl.ds(base + r * _W, _W)], lsem)
            return carry
        lax.fori_loop(0, _ROWS_PER_W, body, 0)

    @pl.when(fld == 0)
    def _():
        lin_rows(inp_hbm)

    @pl.when(fld == 1)
    def _():
        lin_rows(tgt_hbm)

    pltpu.make_async_copy(lin_hbm.at[pl.ds(0, _ROWS_PER_W * _W)],
                          lin_hbm.at[pl.ds(0, _ROWS_PER_W * _W)],
                          lsem).wait()
    i1.wait()
    i2.wait()
    i3.wait()
    i4.wait()
    plsc.subcore_barrier()

    def fire(i, carry):
        sl = pl.ds(i * _CHUNK, _CHUNK)
        pltpu.async_copy(lin_hbm.at[ia_v.at[sl]], vab_v.at[sl], gsem)
        pltpu.async_copy(lin_hbm.at[ib_v.at[sl]],
                         vab_v.at[pl.ds(_PER_W + i * _CHUNK, _CHUNK)], gsem)
        return carry

    lax.fori_loop(0, _CHUNKS, fire, 0)
    pltpu.make_async_copy(lin_hbm.at[pl.ds(0, 2 * _PER_W)], vab_v, gsem).wait()

    def matched_body(k, acc):
        a = vab_v[pl.ds(k * _LANES, _LANES)]
        bb = vab_v[pl.ds(_PER_W + k * _LANES, _LANES)]
        d = _sigmoid16(a) - bb
        return acc + 2.0 * (d * d)

    acc = lax.fori_loop(0, _M_PER_W // _LANES, matched_body,
                        jnp.zeros((_LANES,), jnp.float32))

    def unm_body(k, acc):
        a = vab_v[pl.ds(k * _LANES, _LANES)]
        bb = vab_v[pl.ds(_PER_W + k * _LANES, _LANES)]
        d = _sigmoid16(a) - _sigmoid16(bb)
        return acc + d * d

    acc = lax.fori_loop(_M_PER_W // _LANES, _PER_W // _LANES, unm_body, acc)

    pl.run_scoped(
        lambda acc_ref: (acc_ref.__setitem__((...,), acc),
                         pltpu.sync_copy(acc_ref, out_hbm.at[wid])),
        pltpu.VMEM((_LANES,), jnp.float32),
    )


_WIDS = np.arange(_NW)
_CC = _WIDS % 2
_SS = _WIDS // 2
_M_SRC = (_SS >= 8).astype(np.int32)
_M_BATCH = 4 * _CC + ((_SS % 8) >> 1)
_M_HALF = _SS & 1
_U_BATCH = 4 * _CC + (_SS >> 2)
_U_QUARTER = _SS & 3


def kernel(input, target, pred_birth_idx, pred_death_idx, tgt_birth_idx,
           tgt_death_idx, unm_birth_idx, unm_death_idx):
    offs = (jnp.arange(_B, dtype=jnp.int32) * _HW)[:, None]

    def flat(idx, base=0):
        return (idx.astype(jnp.int32) + offs) + base

    def route_matched(birth, death, base=0):
        stacked = jnp.stack([flat(birth, base).reshape(_B, 2, _M_PER_W),
                             flat(death, base).reshape(_B, 2, _M_PER_W)])
        return stacked[_M_SRC, _M_BATCH, _M_HALF].reshape(-1)

    def route_unm(idx):
        r = flat(idx).reshape(_B, 4, _U_PER_W)
        return r[_U_BATCH, _U_QUARTER].reshape(-1)

    am = route_matched(pred_birth_idx, pred_death_idx)
    bm = route_matched(tgt_birth_idx, tgt_death_idx, _B * _HW)
    au = route_unm(unm_birth_idx)
    bu = route_unm(unm_death_idx)

    partials, _ = _bm_loss_sc(input, target, am, bm, au, bu)
    return jnp.sum(partials).reshape(1)


# speedup vs baseline: 11.7228x; 11.7228x over previous
import functools

import numpy as np

import jax
import jax.numpy as jnp
from jax import lax
from jax.experimental import pallas as pl
from jax.experimental.pallas import tpu as pltpu
from jax.experimental.pallas import tpu_sc as plsc

_B, _H, _W = 8, 512, 512
_HW = _H * _W
_N_MATCHED = 4096
_N_UNMATCHED = 1024

_NW = 32
_CHUNK = 128
_LANES = 16
_LIN = 2 * _B * _HW

_N_M = 2 * _B * _N_MATCHED
_N_U = _B * _N_UNMATCHED
_M_PER_W = _N_M // _NW
_U_PER_W = _N_U // _NW
_PER_W = _M_PER_W + _U_PER_W
_CHUNKS = _PER_W // _CHUNK

_ROWS_PER_W = 2 * _B * _H // _NW


def _sigmoid16(x):
    return 1.0 / (1.0 + jnp.exp(-x))


@functools.partial(
    pl.kernel,
    out_type=(jax.ShapeDtypeStruct((_NW, _LANES), jnp.float32),
              jax.ShapeDtypeStruct((_LIN,), jnp.float32)),
    mesh=plsc.VectorSubcoreMesh(core_axis_name="c", subcore_axis_name="s"),
    scratch_types=[
        pltpu.VMEM((_PER_W,), jnp.int32),
        pltpu.VMEM((_PER_W,), jnp.int32),
        pltpu.VMEM((2 * _PER_W,), jnp.float32),
        pltpu.VMEM((64, _W), jnp.float32),
        pltpu.VMEM((64, _W), jnp.float32),
        pltpu.SemaphoreType.DMA,
        pltpu.SemaphoreType.DMA,
        pltpu.SemaphoreType.DMA,
        pltpu.SemaphoreType.DMA,
    ],
)
def _bm_loss_sc(inp_hbm, tgt_hbm, am_hbm, bm_hbm, au_hbm, bu_hbm,
                out_hbm, lin_hbm, ia_v, ib_v, vab_v, sb0, sb1,
                isem, gsem, rsem, wsem):
    c = lax.axis_index("c")
    s = lax.axis_index("s")
    wid = s * 2 + c

    i1 = pltpu.async_copy(am_hbm.at[pl.ds(wid * _M_PER_W, _M_PER_W)],
                          ia_v.at[pl.ds(0, _M_PER_W)], isem)
    i2 = pltpu.async_copy(bm_hbm.at[pl.ds(wid * _M_PER_W, _M_PER_W)],
                          ib_v.at[pl.ds(0, _M_PER_W)], isem)
    i3 = pltpu.async_copy(au_hbm.at[pl.ds(wid * _U_PER_W, _U_PER_W)],
                          ia_v.at[pl.ds(_M_PER_W, _U_PER_W)], isem)
    i4 = pltpu.async_copy(bu_hbm.at[pl.ds(wid * _U_PER_W, _U_PER_W)],
                          ib_v.at[pl.ds(_M_PER_W, _U_PER_W)], isem)

    fld = s >> 3
    b = 4 * c + ((s >> 1) & 3)
    r0 = (s & 1) * (_H // 2)
    base = fld * (_B * _HW) + b * _HW + r0 * _W

    def lin_rows(src_ref):
        bufs = (sb0, sb1)
        reads = [pltpu.async_copy(
            src_ref.at[b, 0, pl.ds(r0 + cc * 64, 64), :], bufs[cc % 2], rsem)
            for cc in range(2)]
        for cc in range(4):
            buf = bufs[cc % 2]
            reads[cc].wait()

            def wbody(r, carry, buf=buf, cc=cc):
                pltpu.async_copy(
                    buf.at[r],
                    lin_hbm.at[pl.ds(base + (cc * 64 + r) * _W, _W)], wsem)
                return carry

            lax.fori_loop(0, 64, wbody, 0)
            if cc + 2 < 4:
                pltpu.make_async_copy(lin_hbm.at[pl.ds(0, 64 * _W)],
                                      lin_hbm.at[pl.ds(0, 64 * _W)],
                                      wsem).wait()
                reads.append(pltpu.async_copy(
                    src_ref.at[b, 0, pl.ds(r0 + (cc + 2) * 64, 64), :],
                    buf, rsem))

    @pl.when(fld == 0)
    def _():
        lin_rows(inp_hbm)

    @pl.when(fld == 1)
    def _():
        lin_rows(tgt_hbm)

    pltpu.make_async_copy(lin_hbm.at[pl.ds(0, 128 * _W)],
                          lin_hbm.at[pl.ds(0, 128 * _W)],
                          wsem).wait()
    i1.wait()
    i2.wait()
    i3.wait()
    i4.wait()
    plsc.subcore_barrier()

    def fire(i, carry):
        sl = pl.ds(i * _CHUNK, _CHUNK)
        pltpu.async_copy(lin_hbm.at[ia_v.at[sl]], vab_v.at[sl], gsem)
        pltpu.async_copy(lin_hbm.at[ib_v.at[sl]],
                         vab_v.at[pl.ds(_PER_W + i * _CHUNK, _CHUNK)], gsem)
        return carry

    lax.fori_loop(0, _CHUNKS, fire, 0)
    pltpu.make_async_copy(lin_hbm.at[pl.ds(0, 2 * _PER_W)], vab_v, gsem).wait()

    def matched_body(k, acc):
        a = vab_v[pl.ds(k * _LANES, _LANES)]
        bb = vab_v[pl.ds(_PER_W + k * _LANES, _LANES)]
        d = _sigmoid16(a) - bb
        return acc + 2.0 * (d * d)

    acc = lax.fori_loop(0, _M_PER_W // _LANES, matched_body,
                        jnp.zeros((_LANES,), jnp.float32))

    def unm_body(k, acc):
        a = vab_v[pl.ds(k * _LANES, _LANES)]
        bb = vab_v[pl.ds(_PER_W + k * _LANES, _LANES)]
        d = _sigmoid16(a) - _sigmoid16(bb)
        return acc + d * d

    acc = lax.fori_loop(_M_PER_W // _LANES, _PER_W // _LANES, unm_body, acc)

    pl.run_scoped(
        lambda acc_ref: (acc_ref.__setitem__((...,), acc),
                         pltpu.sync_copy(acc_ref, out_hbm.at[wid])),
        pltpu.VMEM((_LANES,), jnp.float32),
    )


_WIDS = np.arange(_NW)
_CC = _WIDS % 2
_SS = _WIDS // 2
_M_SRC = (_SS >= 8).astype(np.int32)
_M_BATCH = 4 * _CC + ((_SS % 8) >> 1)
_M_HALF = _SS & 1
_U_BATCH = 4 * _CC + (_SS >> 2)
_U_QUARTER = _SS & 3


def kernel(input, target, pred_birth_idx, pred_death_idx, tgt_birth_idx,
           tgt_death_idx, unm_birth_idx, unm_death_idx):
    offs = (jnp.arange(_B, dtype=jnp.int32) * _HW)[:, None]

    def flat(idx, base=0):
        return (idx.astype(jnp.int32) + offs) + base

    def route_matched(birth, death, base=0):
        stacked = jnp.stack([flat(birth, base).reshape(_B, 2, _M_PER_W),
                             flat(death, base).reshape(_B, 2, _M_PER_W)])
        return stacked[_M_SRC, _M_BATCH, _M_HALF].reshape(-1)

    def route_unm(idx):
        r = flat(idx).reshape(_B, 4, _U_PER_W)
        return r[_U_BATCH, _U_QUARTER].reshape(-1)

    am = route_matched(pred_birth_idx, pred_death_idx)
    bm = route_matched(tgt_birth_idx, tgt_death_idx, _B * _HW)
    au = route_unm(unm_birth_idx)
    bu = route_unm(unm_death_idx)

    partials, _ = _bm_loss_sc(input, target, am, bm, au, bu)
    return jnp.sum(partials).reshape(1)
